# baseline (device time: 33352 ns/iter reference)
import jax
import jax.numpy as jnp
from jax import lax
from jax.experimental import pallas as pl
from jax.experimental.pallas import tpu as pltpu

N_DEV = 16


def kernel(x, Wq, K_ext, V_ext, Wo):
    B, Sq, Din = x.shape
    _, Skv, Hl, Dh = K_ext.shape
    Dout = Wo.shape[1]
    F = Hl * Dh
    R = B * Sq
    SEG = R // N_DEV

    def body(x_ref, wq_ref, k_ref, v_ref, wo_ref, out_ref,
             recv_ref, recv2_ref, p1_send_sems, p1_recv_sems,
             p2_send_sems, p2_recv_sems):
        my = lax.axis_index("i")

        barrier = pltpu.get_barrier_semaphore()
        for j in range(1, N_DEV):
            pl.semaphore_signal(
                barrier, inc=1,
                device_id=(lax.rem(my + j, N_DEV),),
                device_id_type=pl.DeviceIdType.MESH,
            )
        pl.semaphore_wait(barrier, N_DEV - 1)

        x2d = x_ref[...].reshape(R, Din)
        wq = wq_ref[:, pl.ds(my * F, F)]
        q = jnp.dot(x2d, wq, preferred_element_type=jnp.float32)
        q4 = q.reshape(B, Sq, Hl, Dh)
        kk = k_ref[...]
        vv = v_ref[...]
        ctx_rows = []
        for b in range(B):
            cols = []
            for h in range(Hl):
                s = jnp.dot(q4[b, :, h, :], kk[b, :, h, :].T,
                            preferred_element_type=jnp.float32) * 0.125
                s = s - jnp.max(s, axis=-1, keepdims=True)
                e = jnp.exp(s)
                w = e / jnp.sum(e, axis=-1, keepdims=True)
                cols.append(jnp.dot(w, vv[b, :, h, :],
                                    preferred_element_type=jnp.float32))
            ctx_rows.append(jnp.concatenate(cols, axis=-1))
        ctx = jnp.stack(ctx_rows, axis=0).reshape(R, F)
        wo = wo_ref[pl.ds(my * F, F), :]
        out_ref[...] = jnp.dot(ctx, wo, preferred_element_type=jnp.float32)

        p1_sends = []
        for j in range(1, N_DEV):
            t = lax.rem(my + j, N_DEV)
            rdma = pltpu.make_async_remote_copy(
                src_ref=out_ref.at[pl.ds(t * SEG, SEG), :],
                dst_ref=recv_ref.at[my],
                send_sem=p1_send_sems.at[t],
                recv_sem=p1_recv_sems.at[my],
                device_id=(t,),
                device_id_type=pl.DeviceIdType.MESH,
            )
            rdma.start()
            p1_sends.append(rdma)
        recv_ref[my] = out_ref[pl.ds(my * SEG, SEG), :]

        for j in range(1, N_DEV):
            src = lax.rem(my + j, N_DEV)
            pltpu.make_async_remote_copy(
                src_ref=out_ref.at[pl.ds(0, SEG), :],
                dst_ref=recv_ref.at[src],
                send_sem=p1_send_sems.at[src],
                recv_sem=p1_recv_sems.at[src],
                device_id=(src,),
                device_id_type=pl.DeviceIdType.MESH,
            ).wait_recv()
        for rdma in p1_sends:
            rdma.wait_send()

        reduced = jnp.sum(recv_ref[...], axis=0)
        out_ref[pl.ds(my * SEG, SEG), :] = reduced

        p2_sends = []
        for j in range(1, N_DEV):
            t = lax.rem(my + j, N_DEV)
            rdma = pltpu.make_async_remote_copy(
                src_ref=out_ref.at[pl.ds(my * SEG, SEG), :],
                dst_ref=recv2_ref.at[my],
                send_sem=p2_send_sems.at[t],
                recv_sem=p2_recv_sems.at[my],
                device_id=(t,),
                device_id_type=pl.DeviceIdType.MESH,
            )
            rdma.start()
            p2_sends.append(rdma)
        for j in range(1, N_DEV):
            src = lax.rem(my + j, N_DEV)
            pltpu.make_async_remote_copy(
                src_ref=out_ref.at[pl.ds(0, SEG), :],
                dst_ref=recv2_ref.at[src],
                send_sem=p2_send_sems.at[src],
                recv_sem=p2_recv_sems.at[src],
                device_id=(src,),
                device_id_type=pl.DeviceIdType.MESH,
            ).wait_recv()
            out_ref[pl.ds(src * SEG, SEG), :] = recv2_ref[src]
        for rdma in p2_sends:
            rdma.wait_send()

    out2d = pl.pallas_call(
        body,
        out_shape=jax.ShapeDtypeStruct((R, Dout), jnp.float32),
        in_specs=[pl.BlockSpec(memory_space=pltpu.VMEM)] * 5,
        out_specs=pl.BlockSpec(memory_space=pltpu.VMEM),
        scratch_shapes=[
            pltpu.VMEM((N_DEV, SEG, Dout), jnp.float32),
            pltpu.VMEM((N_DEV, SEG, Dout), jnp.float32),
            pltpu.SemaphoreType.DMA((N_DEV,)),
            pltpu.SemaphoreType.DMA((N_DEV,)),
            pltpu.SemaphoreType.DMA((N_DEV,)),
            pltpu.SemaphoreType.DMA((N_DEV,)),
        ],
        compiler_params=pltpu.CompilerParams(collective_id=0),
    )(x, Wq, K_ext, V_ext, Wo)
    return out2d.reshape(B, Sq, Dout)


# device time: 32605 ns/iter; 1.0229x vs baseline; 1.0229x over previous
import jax
import jax.numpy as jnp
from jax import lax
from jax.experimental import pallas as pl
from jax.experimental.pallas import tpu as pltpu

N_DEV = 16


def kernel(x, Wq, K_ext, V_ext, Wo):
    B, Sq, Din = x.shape
    _, Skv, Hl, Dh = K_ext.shape
    Dout = Wo.shape[1]
    F = Hl * Dh
    R = B * Sq
    SEG = R // N_DEV

    def body(x_ref, wq_ref, k_ref, v_ref, wo_ref, out_ref,
             recv_ref, recv2_ref, p1_send_sems, p1_recv_sems,
             p2_send_sems, p2_recv_sems):
        my = lax.axis_index("i")

        barrier = pltpu.get_barrier_semaphore()
        for j in range(1, N_DEV):
            pl.semaphore_signal(
                barrier, inc=1,
                device_id=(lax.rem(my + j, N_DEV),),
                device_id_type=pl.DeviceIdType.MESH,
            )
        pl.semaphore_wait(barrier, N_DEV - 1)

        wq = wq_ref[:, pl.ds(my * F, F)]
        q = jnp.dot(x_ref[...], wq, preferred_element_type=jnp.float32)
        ctx_rows = []
        for b in range(B):
            kb = k_ref[b]
            vb = v_ref[b]
            cols = []
            for h in range(Hl):
                qs = q[b * Sq:(b + 1) * Sq, h * Dh:(h + 1) * Dh]
                s = lax.dot_general(
                    qs, kb[:, h * Dh:(h + 1) * Dh],
                    (((1,), (1,)), ((), ())),
                    preferred_element_type=jnp.float32)
                e = jnp.exp(s * 0.125)
                w = e / jnp.sum(e, axis=-1, keepdims=True)
                cols.append(jnp.dot(w, vb[:, h * Dh:(h + 1) * Dh],
                                    preferred_element_type=jnp.float32))
            ctx_rows.append(jnp.concatenate(cols, axis=-1))
        ctx = jnp.concatenate(ctx_rows, axis=0)
        wo = wo_ref[pl.ds(my * F, F), :]
        out_ref[...] = jnp.dot(ctx, wo, preferred_element_type=jnp.float32)

        p1_sends = []
        for j in range(1, N_DEV):
            t = lax.rem(my + j, N_DEV)
            rdma = pltpu.make_async_remote_copy(
                src_ref=out_ref.at[pl.ds(t * SEG, SEG), :],
                dst_ref=recv_ref.at[my],
                send_sem=p1_send_sems.at[t],
                recv_sem=p1_recv_sems.at[my],
                device_id=(t,),
                device_id_type=pl.DeviceIdType.MESH,
            )
            rdma.start()
            p1_sends.append(rdma)
        recv_ref[my] = out_ref[pl.ds(my * SEG, SEG), :]

        for j in range(1, N_DEV):
            src = lax.rem(my + j, N_DEV)
            pltpu.make_async_remote_copy(
                src_ref=out_ref.at[pl.ds(0, SEG), :],
                dst_ref=recv_ref.at[src],
                send_sem=p1_send_sems.at[src],
                recv_sem=p1_recv_sems.at[src],
                device_id=(src,),
                device_id_type=pl.DeviceIdType.MESH,
            ).wait_recv()
        for rdma in p1_sends:
            rdma.wait_send()

        reduced = jnp.sum(recv_ref[...], axis=0)
        out_ref[pl.ds(my * SEG, SEG), :] = reduced

        p2_sends = []
        for j in range(1, N_DEV):
            t = lax.rem(my + j, N_DEV)
            rdma = pltpu.make_async_remote_copy(
                src_ref=out_ref.at[pl.ds(my * SEG, SEG), :],
                dst_ref=recv2_ref.at[my],
                send_sem=p2_send_sems.at[t],
                recv_sem=p2_recv_sems.at[my],
                device_id=(t,),
                device_id_type=pl.DeviceIdType.MESH,
            )
            rdma.start()
            p2_sends.append(rdma)
        for j in range(1, N_DEV):
            src = lax.rem(my + j, N_DEV)
            pltpu.make_async_remote_copy(
                src_ref=out_ref.at[pl.ds(0, SEG), :],
                dst_ref=recv2_ref.at[src],
                send_sem=p2_send_sems.at[src],
                recv_sem=p2_recv_sems.at[src],
                device_id=(src,),
                device_id_type=pl.DeviceIdType.MESH,
            ).wait_recv()
            out_ref[pl.ds(src * SEG, SEG), :] = recv2_ref[src]
        for rdma in p2_sends:
            rdma.wait_send()

    out2d = pl.pallas_call(
        body,
        out_shape=jax.ShapeDtypeStruct((R, Dout), jnp.float32),
        in_specs=[pl.BlockSpec(memory_space=pltpu.VMEM)] * 5,
        out_specs=pl.BlockSpec(memory_space=pltpu.VMEM),
        scratch_shapes=[
            pltpu.VMEM((N_DEV, SEG, Dout), jnp.float32),
            pltpu.VMEM((N_DEV, SEG, Dout), jnp.float32),
            pltpu.SemaphoreType.DMA((N_DEV,)),
            pltpu.SemaphoreType.DMA((N_DEV,)),
            pltpu.SemaphoreType.DMA((N_DEV,)),
            pltpu.SemaphoreType.DMA((N_DEV,)),
        ],
        compiler_params=pltpu.CompilerParams(collective_id=0),
    )(x.reshape(R, Din), Wq, K_ext.reshape(B, Skv, F),
      V_ext.reshape(B, Skv, F), Wo)
    return out2d.reshape(B, Sq, Dout)
